# Initial kernel scaffold; baseline (speedup 1.0000x reference)
import jax
import jax.numpy as jnp
from jax.experimental import pallas as pl


def kernel(x, A, S, emb, Wu1, Wv1, Wa1, Wb1, Wc1, Wu2, Wv2, Wa2, Wb2, Wc2):
    raise NotImplementedError("write your pallas kernel here")


# speedup vs baseline: 2.1277x; 2.1277x over previous
import functools

import jax
import jax.numpy as jnp
from jax.experimental import pallas as pl

V, FIN, H = 512, 32, 64
TI = 32
NT = V // TI
TI3 = 16
NT3 = V // TI3
HIGH = jax.lax.Precision.HIGHEST


def _e1_tile(i, x_ref, s_ref, emb_ref, wa1_ref, wb1_ref, wc1_ref, ti):
    xf = x_ref[...]
    xt = x_ref[pl.ds(i * ti, ti), :]
    a1t = jnp.dot(xt, wa1_ref[...], precision=HIGH)
    b1 = jnp.dot(xf, wb1_ref[...], precision=HIGH)
    c = jnp.dot(emb_ref[...], wc1_ref[...], precision=HIGH)
    c0 = c[0:1, :]
    cd = c[1:2, :] - c0
    sv = s_ref[...].astype(jnp.float32)
    e1 = (a1t[:, None, :] + (b1 + c0)[None, :, :]
          + sv[:, :, None] * cd[None, :, :])
    return e1


def _accum_stats(i, stats_ref, e):
    es = jnp.sum(e, axis=(0, 1))[None, :]
    eq = jnp.sum(e * e, axis=(0, 1))[None, :]
    upd = jnp.concatenate([es, eq, jnp.zeros((6, H), jnp.float32)], axis=0)

    @pl.when(i == 0)
    def _():
        stats_ref[...] = upd

    @pl.when(i > 0)
    def _():
        stats_ref[...] = stats_ref[...] + upd


def _pass1(x_ref, s_ref, a_ref, emb_ref, wa1_ref, wb1_ref, wc1_ref, wv1_ref,
           num_ref, den_ref, raw1_ref):
    i = pl.program_id(0)
    e1 = _e1_tile(i, x_ref, s_ref, emb_ref, wa1_ref, wb1_ref, wc1_ref, TI)
    v1 = jnp.dot(x_ref[...], wv1_ref[...], precision=HIGH)
    adj = jnp.minimum(a_ref[...] + s_ref[...], 1).astype(jnp.float32)
    g = jax.nn.sigmoid(e1)
    ga = g * adj[:, :, None]
    num_ref[...] = jnp.sum(ga * v1[None, :, :], axis=1)
    den_ref[...] = jnp.sum(ga, axis=1)
    _accum_stats(i, raw1_ref, e1)


def _prepA(x_ref, wu1_ref, num_ref, den_ref, raw1_ref,
           wa2_ref, wb2_ref, wv2_ref, wu2_ref,
           x1_ref, a2_ref, b2_ref, v2_ref, u2_ref, st1_ref):
    xf = x_ref[...]
    ux1 = jnp.dot(xf, wu1_ref[...], precision=HIGH)
    h = ux1 + num_ref[...] / (den_ref[...] + 1e-6)
    mu = jnp.mean(h, axis=0, keepdims=True)
    var = jnp.mean((h - mu) ** 2, axis=0, keepdims=True)
    x1 = jax.nn.relu((h - mu) / jnp.sqrt(var + 1e-5))
    x1_ref[...] = x1
    a2_ref[...] = jnp.dot(x1, wa2_ref[...], precision=HIGH)
    b2_ref[...] = jnp.dot(x1, wb2_ref[...], precision=HIGH)
    v2_ref[...] = jnp.dot(x1, wv2_ref[...], precision=HIGH)
    u2_ref[...] = jnp.dot(x1, wu2_ref[...], precision=HIGH)
    n = jnp.float32(V) * jnp.float32(V)
    mu1 = raw1_ref[0:1, :] / n
    var1 = raw1_ref[1:2, :] / n - mu1 * mu1
    inv1 = jax.lax.rsqrt(var1 + 1e-5)
    st1_ref[...] = jnp.concatenate(
        [mu1, inv1, jnp.zeros((6, H), jnp.float32)], axis=0)


def _s1_e2_tile(i, x_ref, s_ref, emb_ref, wa1_ref, wb1_ref, wc1_ref,
                st1_ref, a2_ref, b2_ref, wc2_ref, ti):
    e1 = _e1_tile(i, x_ref, s_ref, emb_ref, wa1_ref, wb1_ref, wc1_ref, ti)
    mu1 = st1_ref[0:1, :][None, :, :]
    inv1 = st1_ref[1:2, :][None, :, :]
    s1 = jax.nn.relu((e1 - mu1) * inv1)
    sc = jnp.dot(s1.reshape(ti * V, H), wc2_ref[...],
                 precision=HIGH).reshape(ti, V, H)
    e2 = a2_ref[...][:, None, :] + b2_ref[...][None, :, :] + sc
    return s1, e2


def _pass2(x_ref, s_ref, a_ref, emb_ref, wa1_ref, wb1_ref, wc1_ref,
           st1_ref, a2_ref, b2_ref, v2_ref, wc2_ref,
           num2_ref, den2_ref, raw2_ref):
    i = pl.program_id(0)
    _, e2 = _s1_e2_tile(i, x_ref, s_ref, emb_ref, wa1_ref, wb1_ref, wc1_ref,
                        st1_ref, a2_ref, b2_ref, wc2_ref, TI)
    adj = jnp.minimum(a_ref[...] + s_ref[...], 1).astype(jnp.float32)
    g = jax.nn.sigmoid(e2)
    ga = g * adj[:, :, None]
    num2_ref[...] = jnp.sum(ga * v2_ref[...][None, :, :], axis=1)
    den2_ref[...] = jnp.sum(ga, axis=1)
    _accum_stats(i, raw2_ref, e2)


def _prepB(u2_ref, num2_ref, den2_ref, raw2_ref, x1_ref,
           x2_ref, st2_ref):
    h = u2_ref[...] + num2_ref[...] / (den2_ref[...] + 1e-6)
    mu = jnp.mean(h, axis=0, keepdims=True)
    var = jnp.mean((h - mu) ** 2, axis=0, keepdims=True)
    x2_ref[...] = x1_ref[...] + jax.nn.relu((h - mu) / jnp.sqrt(var + 1e-5))
    n = jnp.float32(V) * jnp.float32(V)
    mu2 = raw2_ref[0:1, :] / n
    var2 = raw2_ref[1:2, :] / n - mu2 * mu2
    inv2 = jax.lax.rsqrt(var2 + 1e-5)
    st2_ref[...] = jnp.concatenate(
        [mu2, inv2, jnp.zeros((6, H), jnp.float32)], axis=0)


def _pass3(x_ref, s_ref, emb_ref, wa1_ref, wb1_ref, wc1_ref,
           st1_ref, a2_ref, b2_ref, wc2_ref, st2_ref,
           s2_ref):
    i = pl.program_id(0)
    s1, e2 = _s1_e2_tile(i, x_ref, s_ref, emb_ref, wa1_ref, wb1_ref, wc1_ref,
                         st1_ref, a2_ref, b2_ref, wc2_ref, TI3)
    mu2 = st2_ref[0:1, :][None, :, :]
    inv2 = st2_ref[1:2, :][None, :, :]
    s2_ref[...] = s1 + jax.nn.relu((e2 - mu2) * inv2)


def _full(shape):
    return pl.BlockSpec(shape, lambda i: tuple(0 for _ in shape))


def _rows(shape):
    return pl.BlockSpec(shape, lambda i: (i,) + tuple(0 for _ in shape[1:]))


@functools.partial(jax.jit, static_argnums=())
def kernel(x, A, S, emb, Wu1, Wv1, Wa1, Wb1, Wc1, Wu2, Wv2, Wa2, Wb2, Wc2):
    x2d = x[0]
    Si = S[0].astype(jnp.int32)
    Ai = A[0].astype(jnp.int32)

    f32 = jnp.float32
    vh = jax.ShapeDtypeStruct((V, H), f32)
    st = jax.ShapeDtypeStruct((8, H), f32)

    num1, den1, raw1 = pl.pallas_call(
        _pass1,
        grid=(NT,),
        in_specs=[_full((V, FIN)), _rows((TI, V)), _rows((TI, V)),
                  _full((2, FIN)), _full((FIN, H)), _full((FIN, H)),
                  _full((FIN, H)), _full((FIN, H))],
        out_specs=[_rows((TI, H)), _rows((TI, H)), _full((8, H))],
        out_shape=[vh, vh, st],
    )(x2d, Si, Ai, emb, Wa1, Wb1, Wc1, Wv1)

    x1, a2, b2, v2, u2, st1 = pl.pallas_call(
        _prepA,
        in_specs=[pl.BlockSpec(memory_space=pl.ANY)] * 0 + [
            _full((V, FIN)), _full((FIN, H)), _full((V, H)), _full((V, H)),
            _full((8, H)), _full((H, H)), _full((H, H)), _full((H, H)),
            _full((H, H))],
        out_specs=[_full((V, H))] * 5 + [_full((8, H))],
        out_shape=[vh, vh, vh, vh, vh, st],
        grid=(1,),
    )(x2d, Wu1, num1, den1, raw1, Wa2, Wb2, Wv2, Wu2)

    num2, den2, raw2 = pl.pallas_call(
        _pass2,
        grid=(NT,),
        in_specs=[_full((V, FIN)), _rows((TI, V)), _rows((TI, V)),
                  _full((2, FIN)), _full((FIN, H)), _full((FIN, H)),
                  _full((FIN, H)), _full((8, H)), _rows((TI, H)),
                  _full((V, H)), _full((V, H)), _full((H, H))],
        out_specs=[_rows((TI, H)), _rows((TI, H)), _full((8, H))],
        out_shape=[vh, vh, st],
    )(x2d, Si, Ai, emb, Wa1, Wb1, Wc1, st1, a2, b2, v2, Wc2)

    x2, st2 = pl.pallas_call(
        _prepB,
        in_specs=[_full((V, H))] * 3 + [_full((8, H)), _full((V, H))],
        out_specs=[_full((V, H)), _full((8, H))],
        out_shape=[vh, st],
        grid=(1,),
    )(u2, num2, den2, raw2, x1)

    s2 = pl.pallas_call(
        _pass3,
        grid=(NT3,),
        in_specs=[_full((V, FIN)), _rows((TI3, V)), _full((2, FIN)),
                  _full((FIN, H)), _full((FIN, H)), _full((FIN, H)),
                  _full((8, H)), _rows((TI3, H)), _full((V, H)),
                  _full((H, H)), _full((8, H))],
        out_specs=[_rows((TI3, V, H))],
        out_shape=[jax.ShapeDtypeStruct((V, V, H), f32)],
    )(x2d, Si, emb, Wa1, Wb1, Wc1, st1, a2, b2, Wc2, st2)[0]

    return (x2[None], s2[None])


# speedup vs baseline: 3.1481x; 1.4795x over previous
import functools

import jax
import jax.numpy as jnp
from jax.experimental import pallas as pl

V, FIN, H = 512, 32, 64
TI = 32
TI2 = 16
TI3 = 16
NT, NT2, NT3 = V // TI, V // TI2, V // TI3
HIGH = jax.lax.Precision.HIGHEST


def _e1_tile(i, x_ref, s_ref, emb_ref, wa1_ref, wb1_ref, wc1_ref, ti):
    xf = x_ref[...]
    xt = x_ref[pl.ds(i * ti, ti), :]
    a1t = jnp.dot(xt, wa1_ref[...], precision=HIGH)
    bc0 = jnp.dot(xf, wb1_ref[...], precision=HIGH)
    c = jnp.dot(emb_ref[...], wc1_ref[...], precision=HIGH)
    c0 = c[0:1, :]
    cd = c[1:2, :] - c0
    bc0 = bc0 + c0
    sv = s_ref[...].astype(jnp.float32)
    e1 = (a1t[:, None, :] + bc0[None, :, :]
          + sv[:, :, None] * cd[None, :, :])
    return e1, a1t, bc0, sv


def _pass1(x_ref, s_ref, a_ref, emb_ref, wa1_ref, wb1_ref, wc1_ref, wv1_ref,
           num_ref, den_ref, raw1_ref):
    i = pl.program_id(0)
    e1, a1t, bc0, sv = _e1_tile(i, x_ref, s_ref, emb_ref, wa1_ref, wb1_ref,
                                wc1_ref, TI)
    v1 = jnp.dot(x_ref[...], wv1_ref[...], precision=HIGH)
    adj = jnp.minimum(a_ref[...] + s_ref[...], 1).astype(jnp.float32)
    g = jax.nn.sigmoid(e1)
    ga = g * adj[:, :, None]
    num_ref[...] = jnp.sum(ga * v1[None, :, :], axis=1)
    den_ref[...] = jnp.sum(ga, axis=1)
    rt = jnp.sum(sv, axis=1, keepdims=True)
    qt = jnp.sum(sv, axis=0, keepdims=True)
    ca = jnp.sum(a1t * rt, axis=0, keepdims=True)
    cb = jnp.dot(qt, bc0, precision=HIGH)
    n1 = jnp.broadcast_to(jnp.sum(rt, axis=0, keepdims=True), (1, H))
    upd = jnp.concatenate([ca, cb, n1, jnp.zeros((5, H), jnp.float32)], axis=0)

    @pl.when(i == 0)
    def _():
        raw1_ref[...] = upd

    @pl.when(i > 0)
    def _():
        raw1_ref[...] = raw1_ref[...] + upd


def _prepA(x_ref, emb_ref, wa1_ref, wb1_ref, wc1_ref, wu1_ref,
           num_ref, den_ref, raw1_ref,
           wa2_ref, wb2_ref, wv2_ref, wu2_ref,
           x1_ref, a2_ref, b2_ref, v2_ref, u2_ref, st1_ref):
    xf = x_ref[...]
    ux1 = jnp.dot(xf, wu1_ref[...], precision=HIGH)
    h = ux1 + num_ref[...] / (den_ref[...] + 1e-6)
    mu = jnp.mean(h, axis=0, keepdims=True)
    var = jnp.mean((h - mu) ** 2, axis=0, keepdims=True)
    x1 = jax.nn.relu((h - mu) / jnp.sqrt(var + 1e-5))
    x1_ref[...] = x1
    a2_ref[...] = jnp.dot(x1, wa2_ref[...], precision=HIGH)
    b2_ref[...] = jnp.dot(x1, wb2_ref[...], precision=HIGH)
    v2_ref[...] = jnp.dot(x1, wv2_ref[...], precision=HIGH)
    u2_ref[...] = jnp.dot(x1, wu2_ref[...], precision=HIGH)
    a1 = jnp.dot(xf, wa1_ref[...], precision=HIGH)
    c = jnp.dot(emb_ref[...], wc1_ref[...], precision=HIGH)
    c0 = c[0:1, :]
    cd = c[1:2, :] - c0
    bc0 = jnp.dot(xf, wb1_ref[...], precision=HIGH) + c0
    sa = jnp.sum(a1, axis=0, keepdims=True)
    sa2 = jnp.sum(a1 * a1, axis=0, keepdims=True)
    sb = jnp.sum(bc0, axis=0, keepdims=True)
    sb2 = jnp.sum(bc0 * bc0, axis=0, keepdims=True)
    ca = raw1_ref[0:1, :]
    cb = raw1_ref[1:2, :]
    n1 = raw1_ref[2:3, :]
    fV = jnp.float32(V)
    n = fV * fV
    se = fV * sa + fV * sb + n1 * cd
    se2 = (fV * sa2 + fV * sb2 + 2.0 * sa * sb
           + 2.0 * cd * (ca + cb) + cd * cd * n1)
    mu1 = se / n
    var1 = se2 / n - mu1 * mu1
    inv1 = jax.lax.rsqrt(var1 + 1e-5)
    st1_ref[...] = jnp.concatenate(
        [mu1, inv1, jnp.zeros((6, H), jnp.float32)], axis=0)


def _s1_tile(i, x_ref, s_ref, emb_ref, wa1_ref, wb1_ref, wc1_ref,
             st1_ref, ti):
    e1, _, _, _ = _e1_tile(i, x_ref, s_ref, emb_ref, wa1_ref, wb1_ref,
                           wc1_ref, ti)
    mu1 = st1_ref[0:1, :][None, :, :]
    inv1 = st1_ref[1:2, :][None, :, :]
    return jax.nn.relu((e1 - mu1) * inv1)


def _pass2(x_ref, s_ref, a_ref, emb_ref, wa1_ref, wb1_ref, wc1_ref,
           st1_ref, a2_ref, b2_ref, v2_ref, wc2_ref,
           num2_ref, den2_ref, raw2_ref, e2_ref):
    i = pl.program_id(0)
    s1 = _s1_tile(i, x_ref, s_ref, emb_ref, wa1_ref, wb1_ref, wc1_ref,
                  st1_ref, TI2)
    sc = jnp.dot(s1.reshape(TI2 * V, H), wc2_ref[...]).reshape(TI2, V, H)
    e2 = a2_ref[...][:, None, :] + b2_ref[...][None, :, :] + sc
    e2_ref[...] = e2
    adj = jnp.minimum(a_ref[...] + s_ref[...], 1).astype(jnp.float32)
    g = jax.nn.sigmoid(e2)
    ga = g * adj[:, :, None]
    num2_ref[...] = jnp.sum(ga * v2_ref[...][None, :, :], axis=1)
    den2_ref[...] = jnp.sum(ga, axis=1)
    es = jnp.sum(e2, axis=(0, 1))[None, :]
    eq = jnp.sum(e2 * e2, axis=(0, 1))[None, :]
    upd = jnp.concatenate([es, eq, jnp.zeros((6, H), jnp.float32)], axis=0)

    @pl.when(i == 0)
    def _():
        raw2_ref[...] = upd

    @pl.when(i > 0)
    def _():
        raw2_ref[...] = raw2_ref[...] + upd


def _prepB(u2_ref, num2_ref, den2_ref, raw2_ref, x1_ref,
           x2_ref, st2_ref):
    h = u2_ref[...] + num2_ref[...] / (den2_ref[...] + 1e-6)
    mu = jnp.mean(h, axis=0, keepdims=True)
    var = jnp.mean((h - mu) ** 2, axis=0, keepdims=True)
    x2_ref[...] = x1_ref[...] + jax.nn.relu((h - mu) / jnp.sqrt(var + 1e-5))
    n = jnp.float32(V) * jnp.float32(V)
    mu2 = raw2_ref[0:1, :] / n
    var2 = raw2_ref[1:2, :] / n - mu2 * mu2
    inv2 = jax.lax.rsqrt(var2 + 1e-5)
    st2_ref[...] = jnp.concatenate(
        [mu2, inv2, jnp.zeros((6, H), jnp.float32)], axis=0)


def _pass3(x_ref, s_ref, emb_ref, wa1_ref, wb1_ref, wc1_ref,
           st1_ref, e2_ref, st2_ref,
           s2_ref):
    i = pl.program_id(0)
    s1 = _s1_tile(i, x_ref, s_ref, emb_ref, wa1_ref, wb1_ref, wc1_ref,
                  st1_ref, TI3)
    e2 = e2_ref[...]
    mu2 = st2_ref[0:1, :][None, :, :]
    inv2 = st2_ref[1:2, :][None, :, :]
    s2_ref[...] = s1 + jax.nn.relu((e2 - mu2) * inv2)


def _full(shape):
    return pl.BlockSpec(shape, lambda i: tuple(0 for _ in shape))


def _rows(shape):
    return pl.BlockSpec(shape, lambda i: (i,) + tuple(0 for _ in shape[1:]))


@functools.partial(jax.jit, static_argnums=())
def kernel(x, A, S, emb, Wu1, Wv1, Wa1, Wb1, Wc1, Wu2, Wv2, Wa2, Wb2, Wc2):
    x2d = x[0]
    Si = S[0].astype(jnp.int32)
    Ai = A[0].astype(jnp.int32)

    f32 = jnp.float32
    vh = jax.ShapeDtypeStruct((V, H), f32)
    st = jax.ShapeDtypeStruct((8, H), f32)

    num1, den1, raw1 = pl.pallas_call(
        _pass1,
        grid=(NT,),
        in_specs=[_full((V, FIN)), _rows((TI, V)), _rows((TI, V)),
                  _full((2, FIN)), _full((FIN, H)), _full((FIN, H)),
                  _full((FIN, H)), _full((FIN, H))],
        out_specs=[_rows((TI, H)), _rows((TI, H)), _full((8, H))],
        out_shape=[vh, vh, st],
    )(x2d, Si, Ai, emb, Wa1, Wb1, Wc1, Wv1)

    x1, a2, b2, v2, u2, st1 = pl.pallas_call(
        _prepA,
        in_specs=[_full((V, FIN)), _full((2, FIN)), _full((FIN, H)),
                  _full((FIN, H)), _full((FIN, H)), _full((FIN, H)),
                  _full((V, H)), _full((V, H)), _full((8, H)),
                  _full((H, H)), _full((H, H)), _full((H, H)),
                  _full((H, H))],
        out_specs=[_full((V, H))] * 5 + [_full((8, H))],
        out_shape=[vh, vh, vh, vh, vh, st],
        grid=(1,),
    )(x2d, emb, Wa1, Wb1, Wc1, Wu1, num1, den1, raw1, Wa2, Wb2, Wv2, Wu2)

    num2, den2, raw2, e2 = pl.pallas_call(
        _pass2,
        grid=(NT2,),
        in_specs=[_full((V, FIN)), _rows((TI2, V)), _rows((TI2, V)),
                  _full((2, FIN)), _full((FIN, H)), _full((FIN, H)),
                  _full((FIN, H)), _full((8, H)), _rows((TI2, H)),
                  _full((V, H)), _full((V, H)), _full((H, H))],
        out_specs=[_rows((TI2, H)), _rows((TI2, H)), _full((8, H)),
                   _rows((TI2, V, H))],
        out_shape=[vh, vh, st, jax.ShapeDtypeStruct((V, V, H), f32)],
    )(x2d, Si, Ai, emb, Wa1, Wb1, Wc1, st1, a2, b2, v2, Wc2)

    x2, st2 = pl.pallas_call(
        _prepB,
        in_specs=[_full((V, H))] * 3 + [_full((8, H)), _full((V, H))],
        out_specs=[_full((V, H)), _full((8, H))],
        out_shape=[vh, st],
        grid=(1,),
    )(u2, num2, den2, raw2, x1)

    s2 = pl.pallas_call(
        _pass3,
        grid=(NT3,),
        in_specs=[_full((V, FIN)), _rows((TI3, V)), _full((2, FIN)),
                  _full((FIN, H)), _full((FIN, H)), _full((FIN, H)),
                  _full((8, H)), _rows((TI3, V, H)), _full((8, H))],
        out_specs=[_rows((TI3, V, H))],
        out_shape=[jax.ShapeDtypeStruct((V, V, H), f32)],
    )(x2d, Si, emb, Wa1, Wb1, Wc1, st1, e2, st2)[0]

    return (x2[None], s2[None])


# speedup vs baseline: 6.2505x; 1.9855x over previous
import functools

import jax
import jax.numpy as jnp
from jax.experimental import pallas as pl

V, FIN, H = 512, 32, 64
TI = 32
TI2 = 16
TI3 = 16
NT, NT2, NT3 = V // TI, V // TI2, V // TI3
HIGH = jax.lax.Precision.HIGHEST


def _e1t_tile(i, x_ref, s_ref, emb_ref, wa1_ref, wb1_ref, wc1_ref, ti):
    xf = x_ref[...]
    xt = x_ref[pl.ds(i * ti, ti), :]
    a1t = jnp.dot(xt, wa1_ref[...], precision=HIGH)
    bc0 = jnp.dot(xf, wb1_ref[...], precision=HIGH)
    c = jnp.dot(emb_ref[...], wc1_ref[...], precision=HIGH)
    c0 = c[0:1, :]
    cd = c[1:2, :] - c0
    bc0 = bc0 + c0
    bc0t = jnp.transpose(bc0)
    cdt = jnp.transpose(cd)
    sv = s_ref[...].astype(jnp.float32)
    e1t = (a1t[:, :, None] + bc0t[None, :, :]
           + sv[:, None, :] * cdt[None, :, :])
    return e1t, a1t, bc0, sv


def _pass1(x_ref, s_ref, a_ref, emb_ref, wa1_ref, wb1_ref, wc1_ref, wv1_ref,
           num_ref, den_ref, raw1_ref):
    i = pl.program_id(0)
    e1t, a1t, bc0, sv = _e1t_tile(i, x_ref, s_ref, emb_ref, wa1_ref, wb1_ref,
                                  wc1_ref, TI)
    v1t = jnp.transpose(jnp.dot(x_ref[...], wv1_ref[...], precision=HIGH))
    adj = jnp.minimum(a_ref[...] + s_ref[...], 1).astype(jnp.float32)
    g = jax.nn.sigmoid(e1t)
    ga = g * adj[:, None, :]
    num_ref[...] = jnp.sum(ga * v1t[None, :, :], axis=2)
    den_ref[...] = jnp.sum(ga, axis=2)
    rt = jnp.sum(sv, axis=1, keepdims=True)
    qt = jnp.sum(sv, axis=0, keepdims=True)
    ca = jnp.sum(a1t * rt, axis=0, keepdims=True)
    cb = jnp.dot(qt, bc0, precision=HIGH)
    n1 = jnp.broadcast_to(jnp.sum(rt, axis=0, keepdims=True), (1, H))
    upd = jnp.concatenate([ca, cb, n1, jnp.zeros((5, H), jnp.float32)], axis=0)

    @pl.when(i == 0)
    def _():
        raw1_ref[...] = upd

    @pl.when(i > 0)
    def _():
        raw1_ref[...] = raw1_ref[...] + upd


def _prepA(x_ref, emb_ref, wa1_ref, wb1_ref, wc1_ref, wu1_ref,
           num_ref, den_ref, raw1_ref,
           wa2_ref, wb2_ref, wv2_ref, wu2_ref,
           x1_ref, a2_ref, b2_ref, v2_ref, u2_ref, st1_ref):
    xf = x_ref[...]
    ux1 = jnp.dot(xf, wu1_ref[...], precision=HIGH)
    h = ux1 + num_ref[...] / (den_ref[...] + 1e-6)
    mu = jnp.mean(h, axis=0, keepdims=True)
    var = jnp.mean((h - mu) ** 2, axis=0, keepdims=True)
    x1 = jax.nn.relu((h - mu) / jnp.sqrt(var + 1e-5))
    x1_ref[...] = x1
    a2_ref[...] = jnp.dot(x1, wa2_ref[...], precision=HIGH)
    b2_ref[...] = jnp.dot(x1, wb2_ref[...], precision=HIGH)
    v2_ref[...] = jnp.dot(x1, wv2_ref[...], precision=HIGH)
    u2_ref[...] = jnp.dot(x1, wu2_ref[...], precision=HIGH)
    a1 = jnp.dot(xf, wa1_ref[...], precision=HIGH)
    c = jnp.dot(emb_ref[...], wc1_ref[...], precision=HIGH)
    c0 = c[0:1, :]
    cd = c[1:2, :] - c0
    bc0 = jnp.dot(xf, wb1_ref[...], precision=HIGH) + c0
    sa = jnp.sum(a1, axis=0, keepdims=True)
    sa2 = jnp.sum(a1 * a1, axis=0, keepdims=True)
    sb = jnp.sum(bc0, axis=0, keepdims=True)
    sb2 = jnp.sum(bc0 * bc0, axis=0, keepdims=True)
    ca = raw1_ref[0:1, :]
    cb = raw1_ref[1:2, :]
    n1 = raw1_ref[2:3, :]
    fV = jnp.float32(V)
    n = fV * fV
    se = fV * sa + fV * sb + n1 * cd
    se2 = (fV * sa2 + fV * sb2 + 2.0 * sa * sb
           + 2.0 * cd * (ca + cb) + cd * cd * n1)
    mu1 = se / n
    var1 = se2 / n - mu1 * mu1
    inv1 = jax.lax.rsqrt(var1 + 1e-5)
    st1_ref[...] = jnp.concatenate(
        [mu1, inv1, jnp.zeros((6, H), jnp.float32)], axis=0)


def _s1t_tile(i, x_ref, s_ref, emb_ref, wa1_ref, wb1_ref, wc1_ref,
              st1_ref, ti):
    e1t, _, _, _ = _e1t_tile(i, x_ref, s_ref, emb_ref, wa1_ref, wb1_ref,
                             wc1_ref, ti)
    mu1t = jnp.transpose(st1_ref[0:1, :])[None, :, :]
    inv1t = jnp.transpose(st1_ref[1:2, :])[None, :, :]
    return jax.nn.relu((e1t - mu1t) * inv1t)


def _pass2(x_ref, s_ref, a_ref, emb_ref, wa1_ref, wb1_ref, wc1_ref,
           st1_ref, a2_ref, b2_ref, v2_ref, wc2_ref,
           num2_ref, den2_ref, raw2_ref, e2_ref):
    i = pl.program_id(0)
    s1t = _s1t_tile(i, x_ref, s_ref, emb_ref, wa1_ref, wb1_ref, wc1_ref,
                    st1_ref, TI2)
    wc2t = jnp.broadcast_to(jnp.transpose(wc2_ref[...])[None, :, :],
                            (TI2, H, H))
    sct = jax.lax.dot_general(
        wc2t, s1t, (((2,), (1,)), ((0,), (0,))))
    b2t = jnp.transpose(b2_ref[...])
    v2t = jnp.transpose(v2_ref[...])
    e2t = a2_ref[...][:, :, None] + b2t[None, :, :] + sct
    e2_ref[...] = e2t
    adj = jnp.minimum(a_ref[...] + s_ref[...], 1).astype(jnp.float32)
    g = jax.nn.sigmoid(e2t)
    ga = g * adj[:, None, :]
    num2_ref[...] = jnp.sum(ga * v2t[None, :, :], axis=2)
    den2_ref[...] = jnp.sum(ga, axis=2)
    es = jnp.transpose(jnp.sum(e2t, axis=(0, 2))[:, None])
    eq = jnp.transpose(jnp.sum(e2t * e2t, axis=(0, 2))[:, None])
    upd = jnp.concatenate([es, eq, jnp.zeros((6, H), jnp.float32)], axis=0)

    @pl.when(i == 0)
    def _():
        raw2_ref[...] = upd

    @pl.when(i > 0)
    def _():
        raw2_ref[...] = raw2_ref[...] + upd


def _prepB(u2_ref, num2_ref, den2_ref, raw2_ref, x1_ref,
           x2_ref, st2_ref):
    h = u2_ref[...] + num2_ref[...] / (den2_ref[...] + 1e-6)
    mu = jnp.mean(h, axis=0, keepdims=True)
    var = jnp.mean((h - mu) ** 2, axis=0, keepdims=True)
    x2_ref[...] = x1_ref[...] + jax.nn.relu((h - mu) / jnp.sqrt(var + 1e-5))
    n = jnp.float32(V) * jnp.float32(V)
    mu2 = raw2_ref[0:1, :] / n
    var2 = raw2_ref[1:2, :] / n - mu2 * mu2
    inv2 = jax.lax.rsqrt(var2 + 1e-5)
    st2_ref[...] = jnp.concatenate(
        [mu2, inv2, jnp.zeros((6, H), jnp.float32)], axis=0)


def _pass3(x_ref, s_ref, emb_ref, wa1_ref, wb1_ref, wc1_ref,
           st1_ref, e2_ref, st2_ref,
           s2_ref):
    i = pl.program_id(0)
    s1t = _s1t_tile(i, x_ref, s_ref, emb_ref, wa1_ref, wb1_ref, wc1_ref,
                    st1_ref, TI3)
    e2t = e2_ref[...]
    mu2t = jnp.transpose(st2_ref[0:1, :])[None, :, :]
    inv2t = jnp.transpose(st2_ref[1:2, :])[None, :, :]
    s2_ref[...] = s1t + jax.nn.relu((e2t - mu2t) * inv2t)


def _full(shape):
    return pl.BlockSpec(shape, lambda i: tuple(0 for _ in shape))


def _rows(shape):
    return pl.BlockSpec(shape, lambda i: (i,) + tuple(0 for _ in shape[1:]))


@functools.partial(jax.jit, static_argnums=())
def kernel(x, A, S, emb, Wu1, Wv1, Wa1, Wb1, Wc1, Wu2, Wv2, Wa2, Wb2, Wc2):
    x2d = x[0]
    Si = S[0].astype(jnp.int32)
    Ai = A[0].astype(jnp.int32)

    f32 = jnp.float32
    vh = jax.ShapeDtypeStruct((V, H), f32)
    st = jax.ShapeDtypeStruct((8, H), f32)
    big = jax.ShapeDtypeStruct((V, H, V), f32)

    num1, den1, raw1 = pl.pallas_call(
        _pass1,
        grid=(NT,),
        in_specs=[_full((V, FIN)), _rows((TI, V)), _rows((TI, V)),
                  _full((2, FIN)), _full((FIN, H)), _full((FIN, H)),
                  _full((FIN, H)), _full((FIN, H))],
        out_specs=[_rows((TI, H)), _rows((TI, H)), _full((8, H))],
        out_shape=[vh, vh, st],
    )(x2d, Si, Ai, emb, Wa1, Wb1, Wc1, Wv1)

    x1, a2, b2, v2, u2, st1 = pl.pallas_call(
        _prepA,
        in_specs=[_full((V, FIN)), _full((2, FIN)), _full((FIN, H)),
                  _full((FIN, H)), _full((FIN, H)), _full((FIN, H)),
                  _full((V, H)), _full((V, H)), _full((8, H)),
                  _full((H, H)), _full((H, H)), _full((H, H)),
                  _full((H, H))],
        out_specs=[_full((V, H))] * 5 + [_full((8, H))],
        out_shape=[vh, vh, vh, vh, vh, st],
        grid=(1,),
    )(x2d, emb, Wa1, Wb1, Wc1, Wu1, num1, den1, raw1, Wa2, Wb2, Wv2, Wu2)

    num2, den2, raw2, e2 = pl.pallas_call(
        _pass2,
        grid=(NT2,),
        in_specs=[_full((V, FIN)), _rows((TI2, V)), _rows((TI2, V)),
                  _full((2, FIN)), _full((FIN, H)), _full((FIN, H)),
                  _full((FIN, H)), _full((8, H)), _rows((TI2, H)),
                  _full((V, H)), _full((V, H)), _full((H, H))],
        out_specs=[_rows((TI2, H)), _rows((TI2, H)), _full((8, H)),
                   _rows((TI2, H, V))],
        out_shape=[vh, vh, st, big],
    )(x2d, Si, Ai, emb, Wa1, Wb1, Wc1, st1, a2, b2, v2, Wc2)

    x2, st2 = pl.pallas_call(
        _prepB,
        in_specs=[_full((V, H))] * 3 + [_full((8, H)), _full((V, H))],
        out_specs=[_full((V, H)), _full((8, H))],
        out_shape=[vh, st],
        grid=(1,),
    )(u2, num2, den2, raw2, x1)

    s2t = pl.pallas_call(
        _pass3,
        grid=(NT3,),
        in_specs=[_full((V, FIN)), _rows((TI3, V)), _full((2, FIN)),
                  _full((FIN, H)), _full((FIN, H)), _full((FIN, H)),
                  _full((8, H)), _rows((TI3, H, V)), _full((8, H))],
        out_specs=[_rows((TI3, H, V))],
        out_shape=[big],
    )(x2d, Si, emb, Wa1, Wb1, Wc1, st1, e2, st2)[0]

    s2 = jnp.transpose(s2t, (0, 2, 1))
    return (x2[None], s2[None])


# speedup vs baseline: 6.9190x; 1.1070x over previous
import functools

import jax
import jax.numpy as jnp
from jax.experimental import pallas as pl

V, FIN, H = 512, 32, 64
TI = 64
TI2 = 32
TI3 = 32
NT, NT2, NT3 = V // TI, V // TI2, V // TI3
HIGH = jax.lax.Precision.HIGHEST


def _e1t_tile(i, x_ref, s_ref, emb_ref, wa1_ref, wb1_ref, wc1_ref, ti):
    xf = x_ref[...]
    xt = x_ref[pl.ds(i * ti, ti), :]
    a1t = jnp.dot(xt, wa1_ref[...], precision=HIGH)
    bc0 = jnp.dot(xf, wb1_ref[...], precision=HIGH)
    c = jnp.dot(emb_ref[...], wc1_ref[...], precision=HIGH)
    c0 = c[0:1, :]
    cd = c[1:2, :] - c0
    bc0 = bc0 + c0
    bc0t = jnp.transpose(bc0)
    cdt = jnp.transpose(cd)
    sv = s_ref[...].astype(jnp.float32)
    e1t = (a1t[:, :, None] + bc0t[None, :, :]
           + sv[:, None, :] * cdt[None, :, :])
    return e1t, a1t, bc0, sv


def _pass1(x_ref, s_ref, a_ref, emb_ref, wa1_ref, wb1_ref, wc1_ref, wv1_ref,
           num_ref, den_ref, raw1_ref):
    i = pl.program_id(0)
    e1t, a1t, bc0, sv = _e1t_tile(i, x_ref, s_ref, emb_ref, wa1_ref, wb1_ref,
                                  wc1_ref, TI)
    v1t = jnp.transpose(jnp.dot(x_ref[...], wv1_ref[...], precision=HIGH))
    adj = jnp.minimum(a_ref[...] + s_ref[...], 1).astype(jnp.float32)
    g = jax.nn.sigmoid(e1t)
    ga = g * adj[:, None, :]
    num_ref[...] = jnp.sum(ga * v1t[None, :, :], axis=2)
    den_ref[...] = jnp.sum(ga, axis=2)
    rt = jnp.sum(sv, axis=1, keepdims=True)
    qt = jnp.sum(sv, axis=0, keepdims=True)
    ca = jnp.sum(a1t * rt, axis=0, keepdims=True)
    cb = jnp.dot(qt, bc0, precision=HIGH)
    n1 = jnp.broadcast_to(jnp.sum(rt, axis=0, keepdims=True), (1, H))
    upd = jnp.concatenate([ca, cb, n1, jnp.zeros((5, H), jnp.float32)], axis=0)

    @pl.when(i == 0)
    def _():
        raw1_ref[...] = upd

    @pl.when(i > 0)
    def _():
        raw1_ref[...] = raw1_ref[...] + upd


def _prepA(x_ref, emb_ref, wa1_ref, wb1_ref, wc1_ref, wu1_ref,
           num_ref, den_ref, raw1_ref,
           wa2_ref, wb2_ref, wv2_ref, wu2_ref,
           x1_ref, a2_ref, b2_ref, v2_ref, u2_ref, st1_ref):
    xf = x_ref[...]
    ux1 = jnp.dot(xf, wu1_ref[...], precision=HIGH)
    h = ux1 + num_ref[...] / (den_ref[...] + 1e-6)
    mu = jnp.mean(h, axis=0, keepdims=True)
    var = jnp.mean((h - mu) ** 2, axis=0, keepdims=True)
    x1 = jax.nn.relu((h - mu) / jnp.sqrt(var + 1e-5))
    x1_ref[...] = x1
    a2_ref[...] = jnp.dot(x1, wa2_ref[...], precision=HIGH)
    b2_ref[...] = jnp.dot(x1, wb2_ref[...], precision=HIGH)
    v2_ref[...] = jnp.dot(x1, wv2_ref[...], precision=HIGH)
    u2_ref[...] = jnp.dot(x1, wu2_ref[...], precision=HIGH)
    a1 = jnp.dot(xf, wa1_ref[...], precision=HIGH)
    c = jnp.dot(emb_ref[...], wc1_ref[...], precision=HIGH)
    c0 = c[0:1, :]
    cd = c[1:2, :] - c0
    bc0 = jnp.dot(xf, wb1_ref[...], precision=HIGH) + c0
    sa = jnp.sum(a1, axis=0, keepdims=True)
    sa2 = jnp.sum(a1 * a1, axis=0, keepdims=True)
    sb = jnp.sum(bc0, axis=0, keepdims=True)
    sb2 = jnp.sum(bc0 * bc0, axis=0, keepdims=True)
    ca = raw1_ref[0:1, :]
    cb = raw1_ref[1:2, :]
    n1 = raw1_ref[2:3, :]
    fV = jnp.float32(V)
    n = fV * fV
    se = fV * sa + fV * sb + n1 * cd
    se2 = (fV * sa2 + fV * sb2 + 2.0 * sa * sb
           + 2.0 * cd * (ca + cb) + cd * cd * n1)
    mu1 = se / n
    var1 = se2 / n - mu1 * mu1
    inv1 = jax.lax.rsqrt(var1 + 1e-5)
    st1_ref[...] = jnp.concatenate(
        [mu1, inv1, jnp.zeros((6, H), jnp.float32)], axis=0)


def _s1t_tile(i, x_ref, s_ref, emb_ref, wa1_ref, wb1_ref, wc1_ref,
              st1_ref, ti):
    e1t, _, _, _ = _e1t_tile(i, x_ref, s_ref, emb_ref, wa1_ref, wb1_ref,
                             wc1_ref, ti)
    mu1t = jnp.transpose(st1_ref[0:1, :])[None, :, :]
    inv1t = jnp.transpose(st1_ref[1:2, :])[None, :, :]
    return jax.nn.relu((e1t - mu1t) * inv1t)


def _pass2(x_ref, s_ref, a_ref, emb_ref, wa1_ref, wb1_ref, wc1_ref,
           st1_ref, a2_ref, b2_ref, v2_ref, wc2_ref,
           num2_ref, den2_ref, raw2_ref, e2_ref):
    i = pl.program_id(0)
    s1t = _s1t_tile(i, x_ref, s_ref, emb_ref, wa1_ref, wb1_ref, wc1_ref,
                    st1_ref, TI2)
    wc2t = jnp.broadcast_to(jnp.transpose(wc2_ref[...])[None, :, :],
                            (TI2, H, H))
    sct = jax.lax.dot_general(
        wc2t, s1t, (((2,), (1,)), ((0,), (0,))))
    b2t = jnp.transpose(b2_ref[...])
    v2t = jnp.transpose(v2_ref[...])
    e2t = a2_ref[...][:, :, None] + b2t[None, :, :] + sct
    e2_ref[...] = e2t.astype(jnp.bfloat16)
    adj = jnp.minimum(a_ref[...] + s_ref[...], 1).astype(jnp.float32)
    g = jax.nn.sigmoid(e2t)
    ga = g * adj[:, None, :]
    num2_ref[...] = jnp.sum(ga * v2t[None, :, :], axis=2)
    den2_ref[...] = jnp.sum(ga, axis=2)
    es = jnp.transpose(jnp.sum(e2t, axis=(0, 2))[:, None])
    eq = jnp.transpose(jnp.sum(e2t * e2t, axis=(0, 2))[:, None])
    upd = jnp.concatenate([es, eq, jnp.zeros((6, H), jnp.float32)], axis=0)

    @pl.when(i == 0)
    def _():
        raw2_ref[...] = upd

    @pl.when(i > 0)
    def _():
        raw2_ref[...] = raw2_ref[...] + upd


def _prepB(u2_ref, num2_ref, den2_ref, raw2_ref, x1_ref,
           x2_ref, st2_ref):
    h = u2_ref[...] + num2_ref[...] / (den2_ref[...] + 1e-6)
    mu = jnp.mean(h, axis=0, keepdims=True)
    var = jnp.mean((h - mu) ** 2, axis=0, keepdims=True)
    x2_ref[...] = x1_ref[...] + jax.nn.relu((h - mu) / jnp.sqrt(var + 1e-5))
    n = jnp.float32(V) * jnp.float32(V)
    mu2 = raw2_ref[0:1, :] / n
    var2 = raw2_ref[1:2, :] / n - mu2 * mu2
    inv2 = jax.lax.rsqrt(var2 + 1e-5)
    st2_ref[...] = jnp.concatenate(
        [mu2, inv2, jnp.zeros((6, H), jnp.float32)], axis=0)


def _pass3(x_ref, s_ref, emb_ref, wa1_ref, wb1_ref, wc1_ref,
           st1_ref, e2_ref, st2_ref,
           s2_ref):
    i = pl.program_id(0)
    s1t = _s1t_tile(i, x_ref, s_ref, emb_ref, wa1_ref, wb1_ref, wc1_ref,
                    st1_ref, TI3)
    e2t = e2_ref[...].astype(jnp.float32)
    mu2t = jnp.transpose(st2_ref[0:1, :])[None, :, :]
    inv2t = jnp.transpose(st2_ref[1:2, :])[None, :, :]
    s2_ref[...] = s1t + jax.nn.relu((e2t - mu2t) * inv2t)


def _full(shape):
    return pl.BlockSpec(shape, lambda i: tuple(0 for _ in shape))


def _rows(shape):
    return pl.BlockSpec(shape, lambda i: (i,) + tuple(0 for _ in shape[1:]))


@functools.partial(jax.jit, static_argnums=())
def kernel(x, A, S, emb, Wu1, Wv1, Wa1, Wb1, Wc1, Wu2, Wv2, Wa2, Wb2, Wc2):
    x2d = x[0]
    Si = S[0].astype(jnp.int32)
    Ai = A[0].astype(jnp.int32)

    f32 = jnp.float32
    vh = jax.ShapeDtypeStruct((V, H), f32)
    st = jax.ShapeDtypeStruct((8, H), f32)
    big = jax.ShapeDtypeStruct((V, H, V), jnp.bfloat16)

    num1, den1, raw1 = pl.pallas_call(
        _pass1,
        grid=(NT,),
        in_specs=[_full((V, FIN)), _rows((TI, V)), _rows((TI, V)),
                  _full((2, FIN)), _full((FIN, H)), _full((FIN, H)),
                  _full((FIN, H)), _full((FIN, H))],
        out_specs=[_rows((TI, H)), _rows((TI, H)), _full((8, H))],
        out_shape=[vh, vh, st],
    )(x2d, Si, Ai, emb, Wa1, Wb1, Wc1, Wv1)

    x1, a2, b2, v2, u2, st1 = pl.pallas_call(
        _prepA,
        in_specs=[_full((V, FIN)), _full((2, FIN)), _full((FIN, H)),
                  _full((FIN, H)), _full((FIN, H)), _full((FIN, H)),
                  _full((V, H)), _full((V, H)), _full((8, H)),
                  _full((H, H)), _full((H, H)), _full((H, H)),
                  _full((H, H))],
        out_specs=[_full((V, H))] * 5 + [_full((8, H))],
        out_shape=[vh, vh, vh, vh, vh, st],
        grid=(1,),
    )(x2d, emb, Wa1, Wb1, Wc1, Wu1, num1, den1, raw1, Wa2, Wb2, Wv2, Wu2)

    num2, den2, raw2, e2 = pl.pallas_call(
        _pass2,
        grid=(NT2,),
        in_specs=[_full((V, FIN)), _rows((TI2, V)), _rows((TI2, V)),
                  _full((2, FIN)), _full((FIN, H)), _full((FIN, H)),
                  _full((FIN, H)), _full((8, H)), _rows((TI2, H)),
                  _full((V, H)), _full((V, H)), _full((H, H))],
        out_specs=[_rows((TI2, H)), _rows((TI2, H)), _full((8, H)),
                   _rows((TI2, H, V))],
        out_shape=[vh, vh, st, big],
    )(x2d, Si, Ai, emb, Wa1, Wb1, Wc1, st1, a2, b2, v2, Wc2)

    x2, st2 = pl.pallas_call(
        _prepB,
        in_specs=[_full((V, H))] * 3 + [_full((8, H)), _full((V, H))],
        out_specs=[_full((V, H)), _full((8, H))],
        out_shape=[vh, st],
        grid=(1,),
    )(u2, num2, den2, raw2, x1)

    s2t = pl.pallas_call(
        _pass3,
        grid=(NT3,),
        in_specs=[_full((V, FIN)), _rows((TI3, V)), _full((2, FIN)),
                  _full((FIN, H)), _full((FIN, H)), _full((FIN, H)),
                  _full((8, H)), _rows((TI3, H, V)), _full((8, H))],
        out_specs=[_rows((TI3, H, V))],
        out_shape=[jax.ShapeDtypeStruct((V, H, V), f32)],
    )(x2d, Si, emb, Wa1, Wb1, Wc1, st1, e2, st2)[0]

    s2 = jnp.transpose(s2t, (0, 2, 1))
    return (x2[None], s2[None])


# speedup vs baseline: 7.0265x; 1.0155x over previous
import functools

import jax
import jax.numpy as jnp
from jax.experimental import pallas as pl
from jax.experimental.pallas import tpu as pltpu

V, FIN, H = 512, 32, 64
TI = 32
NT = V // TI
P1, PA, P2, PB, P3 = 0, NT, NT + 1, 2 * NT + 1, 2 * NT + 2
STEPS = 3 * NT + 2
HIGH = jax.lax.Precision.HIGHEST


def _e1t(row0, x_ref, sv, wa1_ref, wb1_ref, wc1_ref, emb_ref):
    xf = x_ref[...]
    xt = x_ref[pl.ds(row0, TI), :]
    a1t = jnp.dot(xt, wa1_ref[...], precision=HIGH)
    bc0 = jnp.dot(xf, wb1_ref[...], precision=HIGH)
    c = jnp.dot(emb_ref[...], wc1_ref[...], precision=HIGH)
    c0 = c[0:1, :]
    cd = c[1:2, :] - c0
    bc0 = bc0 + c0
    bc0t = jnp.transpose(bc0)
    cdt = jnp.transpose(cd)
    e1t = (a1t[:, :, None] + bc0t[None, :, :]
           + sv[:, None, :] * cdt[None, :, :])
    return e1t, a1t, bc0


def _s1t(row0, x_ref, sv, wa1_ref, wb1_ref, wc1_ref, emb_ref, st1_scr):
    e1t, _, _ = _e1t(row0, x_ref, sv, wa1_ref, wb1_ref, wc1_ref, emb_ref)
    mu1t = jnp.transpose(st1_scr[0:1, :])[None, :, :]
    inv1t = jnp.transpose(st1_scr[1:2, :])[None, :, :]
    return jax.nn.relu((e1t - mu1t) * inv1t)


def _norm_relu(h):
    mu = jnp.mean(h, axis=0, keepdims=True)
    var = jnp.mean((h - mu) ** 2, axis=0, keepdims=True)
    return jax.nn.relu((h - mu) / jnp.sqrt(var + 1e-5))


def _mega(x_ref, s_ref, a_ref, emb_ref,
          wa1_ref, wb1_ref, wc1_ref, wv1_ref, wu1_ref,
          wa2_ref, wb2_ref, wv2_ref, wu2_ref, wc2_ref,
          x2_ref, s2_ref,
          num_scr, den_scr, raw1_scr, raw2_scr,
          x1_scr, a2_scr, b2_scr, v2_scr, u2_scr,
          st1_scr, st2_scr, e2_scr):
    i = pl.program_id(0)

    @pl.when(i < PA)
    def _phase1():
        li = i
        row0 = li * TI
        sv = s_ref[...].astype(jnp.float32)
        e1t, a1t, bc0 = _e1t(row0, x_ref, sv, wa1_ref, wb1_ref, wc1_ref,
                             emb_ref)
        v1t = jnp.transpose(
            jnp.dot(x_ref[...], wv1_ref[...], precision=HIGH))
        adj = jnp.minimum(a_ref[...] + s_ref[...], 1).astype(jnp.float32)
        g = jax.nn.sigmoid(e1t)
        ga = g * adj[:, None, :]
        num_scr[pl.ds(row0, TI), :] = jnp.sum(ga * v1t[None, :, :], axis=2)
        den_scr[pl.ds(row0, TI), :] = jnp.sum(ga, axis=2)
        rt = jnp.sum(sv, axis=1, keepdims=True)
        qt = jnp.sum(sv, axis=0, keepdims=True)
        ca = jnp.sum(a1t * rt, axis=0, keepdims=True)
        cb = jnp.dot(qt, bc0, precision=HIGH)
        n1 = jnp.broadcast_to(jnp.sum(rt, axis=0, keepdims=True), (1, H))
        upd = jnp.concatenate(
            [ca, cb, n1, jnp.zeros((5, H), jnp.float32)], axis=0)

        @pl.when(li == 0)
        def _():
            raw1_scr[...] = upd

        @pl.when(li > 0)
        def _():
            raw1_scr[...] = raw1_scr[...] + upd

    @pl.when(i == PA)
    def _prep_a():
        xf = x_ref[...]
        ux1 = jnp.dot(xf, wu1_ref[...], precision=HIGH)
        x1 = _norm_relu(ux1 + num_scr[...] / (den_scr[...] + 1e-6))
        x1_scr[...] = x1
        a2_scr[...] = jnp.dot(x1, wa2_ref[...], precision=HIGH)
        b2_scr[...] = jnp.dot(x1, wb2_ref[...], precision=HIGH)
        v2_scr[...] = jnp.dot(x1, wv2_ref[...], precision=HIGH)
        u2_scr[...] = jnp.dot(x1, wu2_ref[...], precision=HIGH)
        a1 = jnp.dot(xf, wa1_ref[...], precision=HIGH)
        c = jnp.dot(emb_ref[...], wc1_ref[...], precision=HIGH)
        c0 = c[0:1, :]
        cd = c[1:2, :] - c0
        bc0 = jnp.dot(xf, wb1_ref[...], precision=HIGH) + c0
        sa = jnp.sum(a1, axis=0, keepdims=True)
        sa2 = jnp.sum(a1 * a1, axis=0, keepdims=True)
        sb = jnp.sum(bc0, axis=0, keepdims=True)
        sb2 = jnp.sum(bc0 * bc0, axis=0, keepdims=True)
        ca = raw1_scr[0:1, :]
        cb = raw1_scr[1:2, :]
        n1 = raw1_scr[2:3, :]
        fV = jnp.float32(V)
        n = fV * fV
        se = fV * sa + fV * sb + n1 * cd
        se2 = (fV * sa2 + fV * sb2 + 2.0 * sa * sb
               + 2.0 * cd * (ca + cb) + cd * cd * n1)
        mu1 = se / n
        var1 = se2 / n - mu1 * mu1
        inv1 = jax.lax.rsqrt(var1 + 1e-5)
        st1_scr[...] = jnp.concatenate(
            [mu1, inv1, jnp.zeros((6, H), jnp.float32)], axis=0)

    @pl.when((i > PA) & (i < PB))
    def _phase2():
        li = i - P2
        row0 = li * TI
        sv = s_ref[...].astype(jnp.float32)
        s1t = _s1t(row0, x_ref, sv, wa1_ref, wb1_ref, wc1_ref, emb_ref,
                   st1_scr)
        wc2t = jnp.broadcast_to(jnp.transpose(wc2_ref[...])[None, :, :],
                                (TI, H, H))
        sct = jax.lax.dot_general(
            wc2t, s1t, (((2,), (1,)), ((0,), (0,))))
        a2t = a2_scr[pl.ds(row0, TI), :]
        b2t = jnp.transpose(b2_scr[...])
        v2t = jnp.transpose(v2_scr[...])
        e2t = a2t[:, :, None] + b2t[None, :, :] + sct
        e2_scr[pl.ds(row0, TI), :, :] = e2t.astype(jnp.bfloat16)
        adj = jnp.minimum(a_ref[...] + s_ref[...], 1).astype(jnp.float32)
        g = jax.nn.sigmoid(e2t)
        ga = g * adj[:, None, :]
        num_scr[pl.ds(row0, TI), :] = jnp.sum(ga * v2t[None, :, :], axis=2)
        den_scr[pl.ds(row0, TI), :] = jnp.sum(ga, axis=2)
        es = jnp.transpose(jnp.sum(e2t, axis=(0, 2))[:, None])
        eq = jnp.transpose(jnp.sum(e2t * e2t, axis=(0, 2))[:, None])
        upd = jnp.concatenate(
            [es, eq, jnp.zeros((6, H), jnp.float32)], axis=0)

        @pl.when(li == 0)
        def _():
            raw2_scr[...] = upd

        @pl.when(li > 0)
        def _():
            raw2_scr[...] = raw2_scr[...] + upd

    @pl.when(i == PB)
    def _prep_b():
        h = u2_scr[...] + num_scr[...] / (den_scr[...] + 1e-6)
        x2_ref[...] = x1_scr[...] + _norm_relu(h)
        n = jnp.float32(V) * jnp.float32(V)
        mu2 = raw2_scr[0:1, :] / n
        var2 = raw2_scr[1:2, :] / n - mu2 * mu2
        inv2 = jax.lax.rsqrt(var2 + 1e-5)
        st2_scr[...] = jnp.concatenate(
            [mu2, inv2, jnp.zeros((6, H), jnp.float32)], axis=0)

    @pl.when(i >= P3)
    def _phase3():
        li = i - P3
        row0 = li * TI
        sv = s_ref[...].astype(jnp.float32)
        s1t = _s1t(row0, x_ref, sv, wa1_ref, wb1_ref, wc1_ref, emb_ref,
                   st1_scr)
        e2t = e2_scr[pl.ds(row0, TI), :, :].astype(jnp.float32)
        mu2t = jnp.transpose(st2_scr[0:1, :])[None, :, :]
        inv2t = jnp.transpose(st2_scr[1:2, :])[None, :, :]
        s2_ref[...] = s1t + jax.nn.relu((e2t - mu2t) * inv2t)


def _tile_idx(i):
    return jnp.where(i < P2, jnp.minimum(i, NT - 1),
                     jnp.where(i < P3, jnp.minimum(i - P2, NT - 1), i - P3))


def _full(shape):
    return pl.BlockSpec(shape, lambda i: tuple(0 for _ in shape))


@functools.partial(jax.jit, static_argnums=())
def kernel(x, A, S, emb, Wu1, Wv1, Wa1, Wb1, Wc1, Wu2, Wv2, Wa2, Wb2, Wc2):
    x2d = x[0]
    Si = S[0].astype(jnp.int32)
    Ai = A[0].astype(jnp.int32)

    f32 = jnp.float32
    rowspec = pl.BlockSpec((TI, V), lambda i: (_tile_idx(i), 0))

    x2, s2t = pl.pallas_call(
        _mega,
        compiler_params=pltpu.CompilerParams(
            vmem_limit_bytes=100 * 1024 * 1024),
        grid=(STEPS,),
        in_specs=[_full((V, FIN)), rowspec, rowspec, _full((2, FIN)),
                  _full((FIN, H)), _full((FIN, H)), _full((FIN, H)),
                  _full((FIN, H)), _full((FIN, H)),
                  _full((H, H)), _full((H, H)), _full((H, H)),
                  _full((H, H)), _full((H, H))],
        out_specs=[_full((V, H)),
                   pl.BlockSpec((TI, H, V),
                                lambda i: (jnp.clip(i - P3, 0, NT - 1), 0, 0))],
        out_shape=[jax.ShapeDtypeStruct((V, H), f32),
                   jax.ShapeDtypeStruct((V, H, V), f32)],
        scratch_shapes=[
            pltpu.VMEM((V, H), f32),
            pltpu.VMEM((V, H), f32),
            pltpu.VMEM((8, H), f32),
            pltpu.VMEM((8, H), f32),
            pltpu.VMEM((V, H), f32),
            pltpu.VMEM((V, H), f32),
            pltpu.VMEM((V, H), f32),
            pltpu.VMEM((V, H), f32),
            pltpu.VMEM((V, H), f32),
            pltpu.VMEM((8, H), f32),
            pltpu.VMEM((8, H), f32),
            pltpu.VMEM((V, H, V), jnp.bfloat16),
        ],
    )(x2d, Si, Ai, emb, Wa1, Wb1, Wc1, Wv1, Wu1, Wa2, Wb2, Wv2, Wu2, Wc2)

    s2 = jnp.transpose(s2t, (0, 2, 1))
    return (x2[None], s2[None])


# speedup vs baseline: 7.4484x; 1.0600x over previous
import functools

import jax
import jax.numpy as jnp
from jax.experimental import pallas as pl
from jax.experimental.pallas import tpu as pltpu

V, FIN, H = 512, 32, 64
TI = 32
NT = V // TI
PA, P2, PB, P3 = NT, NT + 1, 2 * NT + 1, 2 * NT + 2
STEPS = 3 * NT + 2
HIGH = jax.lax.Precision.HIGHEST


def _e1t(row0, sv, a1_scr, bc0t_scr, cm_scr):
    a1t = a1_scr[pl.ds(row0, TI), :]
    cdt = cm_scr[:, 0:1]
    return (a1t[:, :, None] + bc0t_scr[...][None, :, :]
            + sv[:, None, :] * cdt[None, :, :])


def _s1t(row0, sv, a1_scr, bc0t_scr, cm_scr, st1t_scr):
    e1t = _e1t(row0, sv, a1_scr, bc0t_scr, cm_scr)
    mu1t = st1t_scr[:, 0:1][None, :, :]
    inv1t = st1t_scr[:, 1:2][None, :, :]
    return jax.nn.relu((e1t - mu1t) * inv1t)


def _norm_relu(h):
    mu = jnp.mean(h, axis=0, keepdims=True)
    var = jnp.mean((h - mu) ** 2, axis=0, keepdims=True)
    return jax.nn.relu((h - mu) / jnp.sqrt(var + 1e-5))


def _mega(x_ref, s_ref, a_ref, emb_ref,
          wa1_ref, wb1_ref, wc1_ref, wv1_ref, wu1_ref,
          wa2_ref, wb2_ref, wv2_ref, wu2_ref, wc2_ref,
          x2_ref, s2_ref,
          num_scr, den_scr, raw1_scr, raw2_scr,
          x1_scr, a2_scr, u2_scr, e2_scr,
          a1_scr, bc0_scr, bc0t_scr, v1t_scr, b2t_scr, v2t_scr,
          cm_scr, st1t_scr, st2t_scr, wc2t_scr):
    i = pl.program_id(0)

    @pl.when(i == 0)
    def _init():
        xf = x_ref[...]
        a1_scr[...] = jnp.dot(xf, wa1_ref[...], precision=HIGH)
        c = jnp.dot(emb_ref[...], wc1_ref[...], precision=HIGH)
        c0 = c[0:1, :]
        cd = c[1:2, :] - c0
        bc0 = jnp.dot(xf, wb1_ref[...], precision=HIGH) + c0
        bc0_scr[...] = bc0
        bc0t_scr[...] = jnp.transpose(bc0)
        v1t_scr[...] = jnp.transpose(
            jnp.dot(xf, wv1_ref[...], precision=HIGH))
        cm_scr[...] = jnp.concatenate(
            [jnp.transpose(cd), jnp.zeros((H, 7), jnp.float32)], axis=1)
        wc2t_scr[...] = jnp.transpose(wc2_ref[...])

    @pl.when(i < PA)
    def _phase1():
        li = i
        row0 = li * TI
        sv = s_ref[...].astype(jnp.float32)
        e1t = _e1t(row0, sv, a1_scr, bc0t_scr, cm_scr)
        adj = jnp.minimum(a_ref[...] + s_ref[...], 1).astype(jnp.float32)
        g = jax.nn.sigmoid(e1t)
        ga = g * adj[:, None, :]
        num_scr[pl.ds(row0, TI), :] = jnp.sum(
            ga * v1t_scr[...][None, :, :], axis=2)
        den_scr[pl.ds(row0, TI), :] = jnp.sum(ga, axis=2)
        rt = jnp.sum(sv, axis=1, keepdims=True)
        qt = jnp.sum(sv, axis=0, keepdims=True)
        ca = jnp.sum(a1_scr[pl.ds(row0, TI), :] * rt, axis=0, keepdims=True)
        cb = jnp.transpose(
            jnp.sum(bc0t_scr[...] * qt, axis=1, keepdims=True))
        n1 = jnp.broadcast_to(jnp.sum(rt, axis=0, keepdims=True), (1, H))
        upd = jnp.concatenate(
            [ca, cb, n1, jnp.zeros((5, H), jnp.float32)], axis=0)

        @pl.when(li == 0)
        def _():
            raw1_scr[...] = upd

        @pl.when(li > 0)
        def _():
            raw1_scr[...] = raw1_scr[...] + upd

    @pl.when(i == PA)
    def _prep_a():
        xf = x_ref[...]
        ux1 = jnp.dot(xf, wu1_ref[...], precision=HIGH)
        x1 = _norm_relu(ux1 + num_scr[...] / (den_scr[...] + 1e-6))
        x1_scr[...] = x1
        a2_scr[...] = jnp.dot(x1, wa2_ref[...], precision=HIGH)
        b2t_scr[...] = jnp.transpose(
            jnp.dot(x1, wb2_ref[...], precision=HIGH))
        v2t_scr[...] = jnp.transpose(
            jnp.dot(x1, wv2_ref[...], precision=HIGH))
        u2_scr[...] = jnp.dot(x1, wu2_ref[...], precision=HIGH)
        a1 = a1_scr[...]
        bc0 = bc0_scr[...]
        cd = jnp.transpose(cm_scr[:, 0:1])
        sa = jnp.sum(a1, axis=0, keepdims=True)
        sa2 = jnp.sum(a1 * a1, axis=0, keepdims=True)
        sb = jnp.sum(bc0, axis=0, keepdims=True)
        sb2 = jnp.sum(bc0 * bc0, axis=0, keepdims=True)
        ca = raw1_scr[0:1, :]
        cb = raw1_scr[1:2, :]
        n1 = raw1_scr[2:3, :]
        fV = jnp.float32(V)
        n = fV * fV
        se = fV * sa + fV * sb + n1 * cd
        se2 = (fV * sa2 + fV * sb2 + 2.0 * sa * sb
               + 2.0 * cd * (ca + cb) + cd * cd * n1)
        mu1 = se / n
        var1 = se2 / n - mu1 * mu1
        inv1 = jax.lax.rsqrt(var1 + 1e-5)
        st1t_scr[...] = jnp.concatenate(
            [jnp.transpose(mu1), jnp.transpose(inv1),
             jnp.zeros((H, 6), jnp.float32)], axis=1)

    @pl.when((i > PA) & (i < PB))
    def _phase2():
        li = i - P2
        row0 = li * TI
        sv = s_ref[...].astype(jnp.float32)
        s1t = _s1t(row0, sv, a1_scr, bc0t_scr, cm_scr, st1t_scr)
        wc2t = jnp.broadcast_to(wc2t_scr[...][None, :, :], (TI, H, H))
        sct = jax.lax.dot_general(
            wc2t, s1t, (((2,), (1,)), ((0,), (0,))))
        a2t = a2_scr[pl.ds(row0, TI), :]
        e2t = a2t[:, :, None] + b2t_scr[...][None, :, :] + sct
        e2_scr[pl.ds(row0, TI), :, :] = e2t.astype(jnp.bfloat16)
        adj = jnp.minimum(a_ref[...] + s_ref[...], 1).astype(jnp.float32)
        g = jax.nn.sigmoid(e2t)
        ga = g * adj[:, None, :]
        num_scr[pl.ds(row0, TI), :] = jnp.sum(
            ga * v2t_scr[...][None, :, :], axis=2)
        den_scr[pl.ds(row0, TI), :] = jnp.sum(ga, axis=2)
        es = jnp.transpose(jnp.sum(e2t, axis=(0, 2))[:, None])
        eq = jnp.transpose(jnp.sum(e2t * e2t, axis=(0, 2))[:, None])
        upd = jnp.concatenate(
            [es, eq, jnp.zeros((6, H), jnp.float32)], axis=0)

        @pl.when(li == 0)
        def _():
            raw2_scr[...] = upd

        @pl.when(li > 0)
        def _():
            raw2_scr[...] = raw2_scr[...] + upd

    @pl.when(i == PB)
    def _prep_b():
        h = u2_scr[...] + num_scr[...] / (den_scr[...] + 1e-6)
        x2_ref[...] = x1_scr[...] + _norm_relu(h)
        n = jnp.float32(V) * jnp.float32(V)
        mu2 = raw2_scr[0:1, :] / n
        var2 = raw2_scr[1:2, :] / n - mu2 * mu2
        inv2 = jax.lax.rsqrt(var2 + 1e-5)
        st2t_scr[...] = jnp.concatenate(
            [jnp.transpose(mu2), jnp.transpose(inv2),
             jnp.zeros((H, 6), jnp.float32)], axis=1)

    @pl.when(i >= P3)
    def _phase3():
        li = i - P3
        row0 = li * TI
        sv = s_ref[...].astype(jnp.float32)
        s1t = _s1t(row0, sv, a1_scr, bc0t_scr, cm_scr, st1t_scr)
        e2t = e2_scr[pl.ds(row0, TI), :, :].astype(jnp.float32)
        mu2t = st2t_scr[:, 0:1][None, :, :]
        inv2t = st2t_scr[:, 1:2][None, :, :]
        s2_ref[...] = s1t + jax.nn.relu((e2t - mu2t) * inv2t)


def _tile_idx(i):
    return jnp.where(i < P2, jnp.minimum(i, NT - 1),
                     jnp.where(i < P3, jnp.minimum(i - P2, NT - 1), i - P3))


def _full(shape):
    return pl.BlockSpec(shape, lambda i: tuple(0 for _ in shape))


@functools.partial(jax.jit, static_argnums=())
def kernel(x, A, S, emb, Wu1, Wv1, Wa1, Wb1, Wc1, Wu2, Wv2, Wa2, Wb2, Wc2):
    x2d = x[0]
    Si = S[0].astype(jnp.int32)
    Ai = A[0].astype(jnp.int32)

    f32 = jnp.float32
    rowspec = pl.BlockSpec((TI, V), lambda i: (_tile_idx(i), 0))

    x2, s2t = pl.pallas_call(
        _mega,
        compiler_params=pltpu.CompilerParams(
            vmem_limit_bytes=100 * 1024 * 1024),
        grid=(STEPS,),
        in_specs=[_full((V, FIN)), rowspec, rowspec, _full((2, FIN)),
                  _full((FIN, H)), _full((FIN, H)), _full((FIN, H)),
                  _full((FIN, H)), _full((FIN, H)),
                  _full((H, H)), _full((H, H)), _full((H, H)),
                  _full((H, H)), _full((H, H))],
        out_specs=[_full((V, H)),
                   pl.BlockSpec((TI, H, V),
                                lambda i: (jnp.clip(i - P3, 0, NT - 1), 0, 0))],
        out_shape=[jax.ShapeDtypeStruct((V, H), f32),
                   jax.ShapeDtypeStruct((V, H, V), f32)],
        scratch_shapes=[
            pltpu.VMEM((V, H), f32),
            pltpu.VMEM((V, H), f32),
            pltpu.VMEM((8, H), f32),
            pltpu.VMEM((8, H), f32),
            pltpu.VMEM((V, H), f32),
            pltpu.VMEM((V, H), f32),
            pltpu.VMEM((V, H), f32),
            pltpu.VMEM((V, H, V), jnp.bfloat16),
            pltpu.VMEM((V, H), f32),
            pltpu.VMEM((V, H), f32),
            pltpu.VMEM((H, V), f32),
            pltpu.VMEM((H, V), f32),
            pltpu.VMEM((H, V), f32),
            pltpu.VMEM((H, V), f32),
            pltpu.VMEM((H, 8), f32),
            pltpu.VMEM((H, 8), f32),
            pltpu.VMEM((H, 8), f32),
            pltpu.VMEM((H, H), f32),
        ],
    )(x2d, Si, Ai, emb, Wa1, Wb1, Wc1, Wv1, Wu1, Wa2, Wb2, Wv2, Wu2, Wc2)

    s2 = jnp.transpose(s2t, (0, 2, 1))
    return (x2[None], s2[None])


# speedup vs baseline: 8.8783x; 1.1920x over previous
import functools

import jax
import jax.numpy as jnp
from jax.experimental import pallas as pl
from jax.experimental.pallas import tpu as pltpu

V, FIN, H = 512, 32, 64
TI = 32
NT = V // TI
PA, P2, PB, P3 = NT, NT + 1, 2 * NT + 1, 2 * NT + 2
STEPS = 3 * NT + 2
HIGH = jax.lax.Precision.HIGHEST


def _e1t(row0, sv, a1_scr, bc0t_scr, cm_scr):
    a1t = a1_scr[pl.ds(row0, TI), :]
    cdt = cm_scr[:, 0:1]
    return (a1t[:, :, None] + bc0t_scr[...][None, :, :]
            + sv[:, None, :] * cdt[None, :, :])


def _s1t(row0, sv, a1_scr, bc0t_scr, cm_scr, st1t_scr):
    e1t = _e1t(row0, sv, a1_scr, bc0t_scr, cm_scr)
    mu1t = st1t_scr[:, 0:1][None, :, :]
    inv1t = st1t_scr[:, 1:2][None, :, :]
    return jax.nn.relu((e1t - mu1t) * inv1t)


def _norm_relu(h):
    mu = jnp.mean(h, axis=0, keepdims=True)
    var = jnp.mean((h - mu) ** 2, axis=0, keepdims=True)
    return jax.nn.relu((h - mu) / jnp.sqrt(var + 1e-5))


def _mega(x_ref, s_ref, a_ref, emb_ref,
          wa1_ref, wb1_ref, wc1_ref, wv1_ref, wu1_ref,
          wa2_ref, wb2_ref, wv2_ref, wu2_ref, wc2_ref,
          x2_ref, s2_ref,
          num_scr, den_scr, raw1_scr, raw2_scr,
          x1_scr, a2_scr, u2_scr, e1s_scr,
          a1_scr, bc0_scr, bc0t_scr, v1t_scr, b2t_scr, v2t_scr,
          cm_scr, st1t_scr, st2t_scr, wc2t_scr):
    i = pl.program_id(0)

    @pl.when(i == 0)
    def _init():
        xf = x_ref[...]
        a1_scr[...] = jnp.dot(xf, wa1_ref[...], precision=HIGH)
        c = jnp.dot(emb_ref[...], wc1_ref[...], precision=HIGH)
        c0 = c[0:1, :]
        cd = c[1:2, :] - c0
        bc0 = jnp.dot(xf, wb1_ref[...], precision=HIGH) + c0
        bc0_scr[...] = bc0
        bc0t_scr[...] = jnp.transpose(bc0)
        v1t_scr[...] = jnp.transpose(
            jnp.dot(xf, wv1_ref[...], precision=HIGH))
        cm_scr[...] = jnp.concatenate(
            [jnp.transpose(cd), jnp.zeros((H, 7), jnp.float32)], axis=1)
        wc2t_scr[...] = jnp.transpose(wc2_ref[...])

    @pl.when(i < PA)
    def _phase1():
        li = i
        row0 = li * TI
        sv = s_ref[...].astype(jnp.float32)
        e1t = _e1t(row0, sv, a1_scr, bc0t_scr, cm_scr)
        e1s_scr[pl.ds(row0, TI), :, :] = e1t.astype(jnp.bfloat16)
        adj = jnp.minimum(a_ref[...] + s_ref[...], 1).astype(jnp.float32)
        g = jax.nn.sigmoid(e1t)
        ga = g * adj[:, None, :]
        num_scr[pl.ds(row0, TI), :] = jnp.sum(
            ga * v1t_scr[...][None, :, :], axis=2)
        den_scr[pl.ds(row0, TI), :] = jnp.sum(ga, axis=2)
        rt = jnp.sum(sv, axis=1, keepdims=True)
        qt = jnp.sum(sv, axis=0, keepdims=True)
        ca = jnp.sum(a1_scr[pl.ds(row0, TI), :] * rt, axis=0, keepdims=True)
        cb = jnp.transpose(
            jnp.sum(bc0t_scr[...] * qt, axis=1, keepdims=True))
        n1 = jnp.broadcast_to(jnp.sum(rt, axis=0, keepdims=True), (1, H))
        upd = jnp.concatenate(
            [ca, cb, n1, jnp.zeros((5, H), jnp.float32)], axis=0)

        @pl.when(li == 0)
        def _():
            raw1_scr[...] = upd

        @pl.when(li > 0)
        def _():
            raw1_scr[...] = raw1_scr[...] + upd

    @pl.when(i == PA)
    def _prep_a():
        xf = x_ref[...]
        ux1 = jnp.dot(xf, wu1_ref[...], precision=HIGH)
        x1 = _norm_relu(ux1 + num_scr[...] / (den_scr[...] + 1e-6))
        x1_scr[...] = x1
        a2_scr[...] = jnp.dot(x1, wa2_ref[...], precision=HIGH)
        b2t_scr[...] = jnp.transpose(
            jnp.dot(x1, wb2_ref[...], precision=HIGH))
        v2t_scr[...] = jnp.transpose(
            jnp.dot(x1, wv2_ref[...], precision=HIGH))
        u2_scr[...] = jnp.dot(x1, wu2_ref[...], precision=HIGH)
        a1 = a1_scr[...]
        bc0 = bc0_scr[...]
        cd = jnp.transpose(cm_scr[:, 0:1])
        sa = jnp.sum(a1, axis=0, keepdims=True)
        sa2 = jnp.sum(a1 * a1, axis=0, keepdims=True)
        sb = jnp.sum(bc0, axis=0, keepdims=True)
        sb2 = jnp.sum(bc0 * bc0, axis=0, keepdims=True)
        ca = raw1_scr[0:1, :]
        cb = raw1_scr[1:2, :]
        n1 = raw1_scr[2:3, :]
        fV = jnp.float32(V)
        n = fV * fV
        se = fV * sa + fV * sb + n1 * cd
        se2 = (fV * sa2 + fV * sb2 + 2.0 * sa * sb
               + 2.0 * cd * (ca + cb) + cd * cd * n1)
        mu1 = se / n
        var1 = se2 / n - mu1 * mu1
        inv1 = jax.lax.rsqrt(var1 + 1e-5)
        st1t_scr[...] = jnp.concatenate(
            [jnp.transpose(mu1), jnp.transpose(inv1),
             jnp.zeros((H, 6), jnp.float32)], axis=1)

    @pl.when((i > PA) & (i < PB))
    def _phase2():
        li = i - P2
        row0 = li * TI
        e1b = e1s_scr[pl.ds(row0, TI), :, :].astype(jnp.float32)
        mu1t = st1t_scr[:, 0:1][None, :, :]
        inv1t = st1t_scr[:, 1:2][None, :, :]
        s1t = jax.nn.relu((e1b - mu1t) * inv1t)
        e1s_scr[pl.ds(row0, TI), :, :] = s1t.astype(jnp.bfloat16)
        wc2t = jnp.broadcast_to(wc2t_scr[...][None, :, :], (TI, H, H))
        sct = jax.lax.dot_general(
            wc2t, s1t, (((2,), (1,)), ((0,), (0,))))
        a2t = a2_scr[pl.ds(row0, TI), :]
        e2t = a2t[:, :, None] + b2t_scr[...][None, :, :] + sct
        adj = jnp.minimum(a_ref[...] + s_ref[...], 1).astype(jnp.float32)
        g = jax.nn.sigmoid(e2t)
        ga = g * adj[:, None, :]
        num_scr[pl.ds(row0, TI), :] = jnp.sum(
            ga * v2t_scr[...][None, :, :], axis=2)
        den_scr[pl.ds(row0, TI), :] = jnp.sum(ga, axis=2)
        es = jnp.transpose(jnp.sum(e2t, axis=(0, 2))[:, None])
        eq = jnp.transpose(jnp.sum(e2t * e2t, axis=(0, 2))[:, None])
        upd = jnp.concatenate(
            [es, eq, jnp.zeros((6, H), jnp.float32)], axis=0)

        @pl.when(li == 0)
        def _():
            raw2_scr[...] = upd

        @pl.when(li > 0)
        def _():
            raw2_scr[...] = raw2_scr[...] + upd

    @pl.when(i == PB)
    def _prep_b():
        h = u2_scr[...] + num_scr[...] / (den_scr[...] + 1e-6)
        x2_ref[...] = x1_scr[...] + _norm_relu(h)
        n = jnp.float32(V) * jnp.float32(V)
        mu2 = raw2_scr[0:1, :] / n
        var2 = raw2_scr[1:2, :] / n - mu2 * mu2
        inv2 = jax.lax.rsqrt(var2 + 1e-5)
        st2t_scr[...] = jnp.concatenate(
            [jnp.transpose(mu2), jnp.transpose(inv2),
             jnp.zeros((H, 6), jnp.float32)], axis=1)

    @pl.when(i >= P3)
    def _phase3():
        li = i - P3
        row0 = li * TI
        s1t = e1s_scr[pl.ds(row0, TI), :, :].astype(jnp.float32)
        wc2t = jnp.broadcast_to(wc2t_scr[...][None, :, :], (TI, H, H))
        sct = jax.lax.dot_general(
            wc2t, s1t, (((2,), (1,)), ((0,), (0,))))
        a2t = a2_scr[pl.ds(row0, TI), :]
        e2t = a2t[:, :, None] + b2t_scr[...][None, :, :] + sct
        mu2t = st2t_scr[:, 0:1][None, :, :]
        inv2t = st2t_scr[:, 1:2][None, :, :]
        s2_ref[...] = s1t + jax.nn.relu((e2t - mu2t) * inv2t)


def _tile_idx(i):
    return jnp.where(i < P2, jnp.minimum(i, NT - 1),
                     jnp.minimum(i - P2, NT - 1))


def _full(shape):
    return pl.BlockSpec(shape, lambda i: tuple(0 for _ in shape))


@functools.partial(jax.jit, static_argnums=())
def kernel(x, A, S, emb, Wu1, Wv1, Wa1, Wb1, Wc1, Wu2, Wv2, Wa2, Wb2, Wc2):
    x2d = x[0]
    Si = S[0].astype(jnp.int32)
    Ai = A[0].astype(jnp.int32)

    f32 = jnp.float32
    rowspec = pl.BlockSpec((TI, V), lambda i: (_tile_idx(i), 0))

    x2, s2t = pl.pallas_call(
        _mega,
        compiler_params=pltpu.CompilerParams(
            vmem_limit_bytes=63 * 1024 * 1024),
        grid=(STEPS,),
        in_specs=[_full((V, FIN)), rowspec, rowspec, _full((2, FIN)),
                  _full((FIN, H)), _full((FIN, H)), _full((FIN, H)),
                  _full((FIN, H)), _full((FIN, H)),
                  _full((H, H)), _full((H, H)), _full((H, H)),
                  _full((H, H)), _full((H, H))],
        out_specs=[_full((V, H)),
                   pl.BlockSpec((TI, H, V),
                                lambda i: (jnp.clip(i - P3, 0, NT - 1), 0, 0))],
        out_shape=[jax.ShapeDtypeStruct((V, H), f32),
                   jax.ShapeDtypeStruct((V, H, V), f32)],
        scratch_shapes=[
            pltpu.VMEM((V, H), f32),
            pltpu.VMEM((V, H), f32),
            pltpu.VMEM((8, H), f32),
            pltpu.VMEM((8, H), f32),
            pltpu.VMEM((V, H), f32),
            pltpu.VMEM((V, H), f32),
            pltpu.VMEM((V, H), f32),
            pltpu.VMEM((V, H, V), jnp.bfloat16),
            pltpu.VMEM((V, H), f32),
            pltpu.VMEM((V, H), f32),
            pltpu.VMEM((H, V), f32),
            pltpu.VMEM((H, V), f32),
            pltpu.VMEM((H, V), f32),
            pltpu.VMEM((H, V), f32),
            pltpu.VMEM((H, 8), f32),
            pltpu.VMEM((H, 8), f32),
            pltpu.VMEM((H, 8), f32),
            pltpu.VMEM((H, H), f32),
        ],
    )(x2d, Si, Ai, emb, Wa1, Wb1, Wc1, Wv1, Wu1, Wa2, Wb2, Wv2, Wu2, Wc2)

    s2 = jnp.transpose(s2t, (0, 2, 1))
    return (x2[None], s2[None])
